# baseline (speedup 1.0000x reference)
import functools

import jax
import jax.numpy as jnp
from jax import lax
from jax.experimental import pallas as pl
from jax.experimental.pallas import tpu as pltpu
from jax.experimental.pallas import tpu_sc as plsc

BATCH = 16384
EMBED_DIM = 64

_info = plsc.get_sparse_core_info()
_NC, _NS = _info.num_cores, _info.num_subcores
_NW = _NC * _NS
_B_PER_W = BATCH // _NW

_mesh = plsc.VectorSubcoreMesh(core_axis_name="c", subcore_axis_name="s")


@functools.partial(
    pl.kernel,
    mesh=_mesh,
    compiler_params=pltpu.CompilerParams(use_tc_tiling_on_sc=False),
    out_type=(
        jax.ShapeDtypeStruct((BATCH, EMBED_DIM), jnp.float32),
        jax.ShapeDtypeStruct((BATCH, EMBED_DIM), jnp.float32),
    ),
    scratch_types=[
        pltpu.VMEM((_B_PER_W,), jnp.int32),
        pltpu.VMEM((_B_PER_W,), jnp.int32),
        pltpu.VMEM((_B_PER_W, EMBED_DIM), jnp.float32),
        pltpu.VMEM((_B_PER_W, EMBED_DIM), jnp.float32),
        pltpu.SemaphoreType.DMA,
        pltpu.SemaphoreType.DMA,
    ],
)
def _two_tower_sc(u_ids, i_ids, user_table, item_table, u_out, i_out,
                  u_idx, i_idx, u_rows, i_rows, u_sem, i_sem):
    wid = lax.axis_index("s") * _NC + lax.axis_index("c")
    base = wid * _B_PER_W
    pltpu.sync_copy(u_ids.at[pl.ds(base, _B_PER_W)], u_idx)
    pltpu.sync_copy(i_ids.at[pl.ds(base, _B_PER_W)], i_idx)
    cu = pltpu.async_copy(user_table.at[u_idx], u_rows, u_sem)
    ci = pltpu.async_copy(item_table.at[i_idx], i_rows, i_sem)
    cu.wait()
    pltpu.sync_copy(u_rows, u_out.at[pl.ds(base, _B_PER_W)])
    ci.wait()
    pltpu.sync_copy(i_rows, i_out.at[pl.ds(base, _B_PER_W)])


def kernel(u_ids, i_ids, user_table, item_table):
    return _two_tower_sc(u_ids, i_ids, user_table, item_table)
